# baseline (speedup 1.0000x reference)
import jax
import jax.numpy as jnp
from jax import lax
from jax.experimental import pallas as pl
from jax.experimental.pallas import tpu as pltpu

_EPS = 1e-5


def _sum_lanes(x):
    hw = x.shape[-1]
    n = hw // 128
    if n == 0:
        return jnp.sum(x, axis=-1, keepdims=True)
    chunks = [x[:, j * 128:(j + 1) * 128] for j in range(n)]
    tail = x[:, n * 128:] if hw % 128 else None
    while len(chunks) > 1:
        nxt = [a + b for a, b in zip(chunks[0::2], chunks[1::2])]
        if len(chunks) % 2:
            nxt.append(chunks[-1])
        chunks = nxt
    pooled = jnp.sum(chunks[0], axis=-1, keepdims=True)
    if tail is not None:
        pooled = pooled + jnp.sum(tail, axis=-1, keepdims=True)
    return pooled


def _fused_body(x_vmem, w_vmem, b_vmem, s_vmem, t_vmem, o_vmem):
    x = x_vmem[0]
    pooled = _sum_lanes(x)

    z = jnp.dot(w_vmem[...], pooled, preferred_element_type=jnp.float32)
    gate = jax.nn.sigmoid((z + b_vmem[...]) * s_vmem[...] + t_vmem[...])

    g = gate * s_vmem[...]
    o_vmem[0] = jnp.maximum(x * g + t_vmem[...], 0.0)


def kernel(x, w, b, gamma, beta, mean, var):
    N, C, H, W = x.shape
    HW = H * W
    xf = x.reshape(N, C, HW).astype(jnp.float32)

    s_vec = gamma * lax.rsqrt(var + _EPS)
    t_vec = beta - mean * s_vec
    s_col = s_vec.reshape(C, 1)
    t_col = t_vec.reshape(C, 1)
    w_scaled = w.astype(jnp.float32) * (1.0 / HW)
    b_col = b.reshape(C, 1).astype(jnp.float32)

    out_init = lax.empty((N, C, HW), jnp.float32)
    n_cores = getattr(jax.devices()[0], "num_cores", 1)
    mesh = pltpu.create_tensorcore_mesh("core", num_cores=n_cores)

    def run(refs):
        x_ref, w_ref, b_ref, s_ref, t_ref, o_ref = refs

        @pl.core_map(
            mesh,
            compiler_params=pltpu.CompilerParams(
                vmem_limit_bytes=32 * 1024 * 1024),
            name="fused_channel_attention",
        )
        def _():
            pltpu.emit_pipeline(
                _fused_body,
                grid=(N,),
                in_specs=[
                    pl.BlockSpec((1, C, HW), lambda i: (i, 0, 0)),
                    pl.BlockSpec((C, C), lambda i: (0, 0)),
                    pl.BlockSpec((C, 1), lambda i: (0, 0)),
                    pl.BlockSpec((C, 1), lambda i: (0, 0)),
                    pl.BlockSpec((C, 1), lambda i: (0, 0)),
                ],
                out_specs=[pl.BlockSpec((1, C, HW), lambda i: (i, 0, 0))],
                core_axis_name="core",
                dimension_semantics=(pltpu.PARALLEL,),
            )(x_ref, w_ref, b_ref, s_ref, t_ref, o_ref)

    _, _, _, _, _, out = pl.run_state(run)(
        (xf, w_scaled, b_col, s_col, t_col, out_init))
    return out.reshape(N, C, H, W)


# speedup vs baseline: 1.0033x; 1.0033x over previous
import jax
import jax.numpy as jnp
from jax import lax
from jax.experimental import pallas as pl
from jax.experimental.pallas import tpu as pltpu

_EPS = 1e-5
_BLOCK_N = 2


def _sum_lanes(x):
    hw = x.shape[-1]
    n = hw // 128
    if n == 0:
        return jnp.sum(x, axis=-1, keepdims=True)
    chunks = [x[:, j * 128:(j + 1) * 128] for j in range(n)]
    tail = x[:, n * 128:] if hw % 128 else None
    while len(chunks) > 1:
        nxt = [a + b for a, b in zip(chunks[0::2], chunks[1::2])]
        if len(chunks) % 2:
            nxt.append(chunks[-1])
        chunks = nxt
    pooled = jnp.sum(chunks[0], axis=-1, keepdims=True)
    if tail is not None:
        pooled = pooled + jnp.sum(tail, axis=-1, keepdims=True)
    return pooled


def _fused_kernel(x_ref, w_ref, b_ref, s_ref, t_ref, o_ref):
    bn = x_ref.shape[0]
    for k in range(bn):
        x = x_ref[k]
        pooled = _sum_lanes(x)
        z = jnp.dot(w_ref[...], pooled,
                    preferred_element_type=jnp.float32)
        gate = jax.nn.sigmoid((z + b_ref[...]) * s_ref[...] + t_ref[...])
        g = gate * s_ref[...]
        o_ref[k] = jnp.maximum(x * g + t_ref[...], 0.0)


def kernel(x, w, b, gamma, beta, mean, var):
    N, C, H, W = x.shape
    HW = H * W
    xf = x.reshape(N, C, HW).astype(jnp.float32)

    s_vec = gamma * lax.rsqrt(var + _EPS)
    t_vec = beta - mean * s_vec
    s_col = s_vec.reshape(C, 1)
    t_col = t_vec.reshape(C, 1)
    w_scaled = w.astype(jnp.float32) * (1.0 / HW)
    b_col = b.reshape(C, 1).astype(jnp.float32)

    bn = _BLOCK_N if N % _BLOCK_N == 0 else 1
    out = pl.pallas_call(
        _fused_kernel,
        out_shape=jax.ShapeDtypeStruct((N, C, HW), jnp.float32),
        grid=(N // bn,),
        in_specs=[
            pl.BlockSpec((bn, C, HW), lambda i: (i, 0, 0)),
            pl.BlockSpec((C, C), lambda i: (0, 0)),
            pl.BlockSpec((C, 1), lambda i: (0, 0)),
            pl.BlockSpec((C, 1), lambda i: (0, 0)),
            pl.BlockSpec((C, 1), lambda i: (0, 0)),
        ],
        out_specs=pl.BlockSpec((bn, C, HW), lambda i: (i, 0, 0)),
        compiler_params=pltpu.CompilerParams(
            dimension_semantics=("arbitrary",),
            vmem_limit_bytes=56 * 1024 * 1024),
        name="fused_channel_attention",
    )(xf, w_scaled, b_col, s_col, t_col)
    return out.reshape(N, C, H, W)
